# baseline (speedup 1.0000x reference)
import functools
import jax
import jax.numpy as jnp
from jax import lax
from jax.experimental import pallas as pl
from jax.experimental.pallas import tpu as pltpu
from jax.experimental.pallas import tpu_sc as plsc

_NW = 32
_CHUNK = 320
_BPC = 16
_STREAMS = 4
_TBLK = 8192


def _widen_tc(WT, V, D):
    grid = (pl.cdiv(V, _TBLK),)

    def body(wt_ref, id_ref, out_ref):
        out_ref[:, :D] = jax.lax.dot_general(
            wt_ref[...], id_ref[...], (((0,), (0,)), ((), ())),
            preferred_element_type=jnp.float32)

    return pl.pallas_call(
        body,
        grid=grid,
        in_specs=[pl.BlockSpec((D, _TBLK), lambda i: (0, i)),
                  pl.BlockSpec((D, D), lambda i: (0, 0))],
        out_specs=pl.BlockSpec((_TBLK, 2 * D), lambda i: (i, 0)),
        out_shape=jax.ShapeDtypeStruct((V, 2 * D), jnp.float32),
    )(WT, jnp.eye(D, dtype=jnp.float32))


def _gather_sc(idx, W128, B, H, D):
    N = B * H
    b_per_w = N // _NW
    n_chunks = b_per_w // _CHUNK
    bs_per_w = B // _NW
    mesh = plsc.VectorSubcoreMesh(core_axis_name="c", subcore_axis_name="s")

    @functools.partial(
        pl.kernel, mesh=mesh,
        out_type=jax.ShapeDtypeStruct((B, H, 2 * D), jnp.float32),
        scratch_types=[
            pltpu.VMEM((b_per_w,), jnp.int32),
            pltpu.VMEM((2, _CHUNK, 2 * D), jnp.float32),
            pltpu.SemaphoreType.DMA,
            pltpu.SemaphoreType.DMA,
            pltpu.SemaphoreType.DMA,
            pltpu.SemaphoreType.DMA,
        ],
    )
    def k(idx_hbm, w_hbm, out_hbm, idx_v, raw_v, g0, g1, s0, s1):
        gsems = (g0, g1)
        ssems = (s0, s1)
        wid = lax.axis_index("s") * 2 + lax.axis_index("c")
        base = wid * b_per_w
        bbase = wid * bs_per_w
        pltpu.sync_copy(idx_hbm.at[pl.ds(pl.multiple_of(base, 8), b_per_w)],
                        idx_v)

        def fire_gather(s, buf):
            for j in range(_STREAMS):
                pltpu.async_copy(
                    w_hbm.at[idx_v.at[pl.ds(s * _CHUNK + j * 80, 80)]],
                    raw_v.at[buf, pl.ds(j * 80, 80)],
                    gsems[buf])

        def drain_gather(s, buf):
            for j in range(_STREAMS):
                pltpu.make_async_copy(
                    w_hbm.at[idx_v.at[pl.ds(s * _CHUNK + j * 80, 80)]],
                    raw_v.at[buf, pl.ds(j * 80, 80)],
                    gsems[buf]).wait()

        def fire_store(s, buf):
            for bi in range(_BPC):
                pltpu.async_copy(
                    raw_v.at[buf, pl.ds(bi * H, H)],
                    out_hbm.at[bbase + s * _BPC + bi],
                    ssems[buf])

        def drain_store(s, buf):
            for bi in range(_BPC):
                pltpu.make_async_copy(
                    raw_v.at[buf, pl.ds(bi * H, H)],
                    out_hbm.at[bbase + s * _BPC + bi],
                    ssems[buf]).wait()

        def step(s, buf, fire, wait_prev):
            drain_gather(s, buf)
            if wait_prev:
                drain_store(s, 1 - buf)
            if fire:
                fire_gather(s + 1, 1 - buf)
            fire_store(s, buf)

        fire_gather(0, 0)
        step(0, 0, True, False)
        step(1, 1, True, True)

        def body(t, c):
            s = 2 * t
            step(s, 0, True, True)
            step(s + 1, 1, True, True)
            return c
        lax.fori_loop(1, n_chunks // 2 - 1, body, 0)

        step(n_chunks - 2, 0, True, True)
        step(n_chunks - 1, 1, False, True)
        drain_store(n_chunks - 1, 1)

    return k(idx, W128)


def kernel(x, W):
    B, H = x.shape
    V, D = W.shape
    W128 = _widen_tc(W.T, V, D)
    idx = x.reshape(B * H).astype(jnp.int32)
    out = _gather_sc(idx, W128, B, H, D)
    return out[:, :, :D]


# speedup vs baseline: 1.0543x; 1.0543x over previous
import functools
import jax
import jax.numpy as jnp
from jax import lax
from jax.experimental import pallas as pl
from jax.experimental.pallas import tpu as pltpu
from jax.experimental.pallas import tpu_sc as plsc

_NW = 32
_CHUNK = 320
_BPC = 16
_STREAMS = 4
_TBLK = 16384


def _widen_tc(WT, V, D):
    grid = (pl.cdiv(V, _TBLK),)

    def body(wt_ref, out_ref):
        out_ref[:, :D] = wt_ref[...].T

    return pl.pallas_call(
        body,
        grid=grid,
        in_specs=[pl.BlockSpec((D, _TBLK), lambda i: (0, i))],
        out_specs=pl.BlockSpec((_TBLK, 2 * D), lambda i: (i, 0)),
        out_shape=jax.ShapeDtypeStruct((V, 2 * D), jnp.float32),
    )(WT)


def _gather_sc(idx, W128, B, H, D):
    N = B * H
    b_per_w = N // _NW
    n_chunks = b_per_w // _CHUNK
    bs_per_w = B // _NW
    mesh = plsc.VectorSubcoreMesh(core_axis_name="c", subcore_axis_name="s")

    @functools.partial(
        pl.kernel, mesh=mesh,
        out_type=jax.ShapeDtypeStruct((B, H, 2 * D), jnp.float32),
        scratch_types=[
            pltpu.VMEM((b_per_w,), jnp.int32),
            pltpu.VMEM((2, _CHUNK, 2 * D), jnp.float32),
            pltpu.SemaphoreType.DMA,
            pltpu.SemaphoreType.DMA,
            pltpu.SemaphoreType.DMA,
            pltpu.SemaphoreType.DMA,
        ],
    )
    def k(idx_hbm, w_hbm, out_hbm, idx_v, raw_v, g0, g1, s0, s1):
        gsems = (g0, g1)
        ssems = (s0, s1)
        wid = lax.axis_index("s") * 2 + lax.axis_index("c")
        base = wid * b_per_w
        bbase = wid * bs_per_w
        pltpu.sync_copy(idx_hbm.at[pl.ds(pl.multiple_of(base, 8), b_per_w)],
                        idx_v)

        def fire_gather(s, buf):
            for j in range(_STREAMS):
                pltpu.async_copy(
                    w_hbm.at[idx_v.at[pl.ds(s * _CHUNK + j * 80, 80)]],
                    raw_v.at[buf, pl.ds(j * 80, 80)],
                    gsems[buf])

        def drain_gather(s, buf):
            for j in range(_STREAMS):
                pltpu.make_async_copy(
                    w_hbm.at[idx_v.at[pl.ds(s * _CHUNK + j * 80, 80)]],
                    raw_v.at[buf, pl.ds(j * 80, 80)],
                    gsems[buf]).wait()

        def fire_store(s, buf):
            for bi in range(_BPC):
                pltpu.async_copy(
                    raw_v.at[buf, pl.ds(bi * H, H)],
                    out_hbm.at[bbase + s * _BPC + bi],
                    ssems[buf])

        def drain_store(s, buf):
            for bi in range(_BPC):
                pltpu.make_async_copy(
                    raw_v.at[buf, pl.ds(bi * H, H)],
                    out_hbm.at[bbase + s * _BPC + bi],
                    ssems[buf]).wait()

        def step(s, buf, fire, wait_prev):
            drain_gather(s, buf)
            if wait_prev:
                drain_store(s, 1 - buf)
            if fire:
                fire_gather(s + 1, 1 - buf)
            fire_store(s, buf)

        fire_gather(0, 0)
        step(0, 0, True, False)
        step(1, 1, True, True)

        def body(t, c):
            s = 2 * t
            step(s, 0, True, True)
            step(s + 1, 1, True, True)
            return c
        lax.fori_loop(1, n_chunks // 2 - 1, body, 0)

        step(n_chunks - 2, 0, True, True)
        step(n_chunks - 1, 1, False, True)
        drain_store(n_chunks - 1, 1)

    return k(idx, W128)


def kernel(x, W):
    B, H = x.shape
    V, D = W.shape
    W128 = _widen_tc(W.T, V, D)
    idx = x.reshape(B * H).astype(jnp.int32)
    out = _gather_sc(idx, W128, B, H, D)
    return out[:, :, :D]
